# baseline (speedup 1.0000x reference)
import functools

import jax
import jax.numpy as jnp
from jax import lax
from jax.experimental import pallas as pl
from jax.experimental.pallas import tpu as pltpu
from jax.experimental.pallas import tpu_sc as plsc

_B = 16384
_D = 64
_DP = 128
_L = 16
_NC = 2
_NS = 16
_NW = _NC * _NS
_TB = _B // _NW
_C = 64
_NCH = _TB // _C


def _vsqrt(x):
    xi = lax.bitcast_convert_type(x, jnp.int32)
    yi = jnp.int32(0x5F3759DF) - (xi >> 1)
    y = lax.bitcast_convert_type(yi, jnp.float32)
    for _ in range(3):
        y = y * (1.5 - 0.5 * x * y * y)
    return x * y


def _signish(x):
    return jnp.minimum(jnp.maximum(x * 1e12, -1.0), 1.0)


def _sc_partials(entity_emb, relation_emb, idx):
    mesh = plsc.VectorSubcoreMesh(core_axis_name="c", subcore_axis_name="s")

    @functools.partial(
        pl.kernel,
        mesh=mesh,
        compiler_params=pltpu.CompilerParams(
            needs_layout_passes=False, use_tc_tiling_on_sc=True
        ),
        out_type=jax.ShapeDtypeStruct((_NW * _L,), jnp.float32),
        scratch_types=[pltpu.VMEM((6, _NCH, _C), jnp.int32)]
        + [pltpu.VMEM((_C, _DP), jnp.float32) for _ in range(12)]
        + [
            pltpu.VMEM((_L,), jnp.float32),
            pltpu.SemaphoreType.DMA,
            pltpu.SemaphoreType.DMA,
        ],
    )
    def k(ent_hbm, rel_hbm, idx_hbm, out_hbm, idxv, *rest):
        rows = [rest[0:6], rest[6:12]]
        lossbuf, sem0, sem1 = rest[12], rest[13], rest[14]
        wid = lax.axis_index("s") * _NC + lax.axis_index("c")
        sems = [sem0, sem1]

        for j in range(6):
            pltpu.sync_copy(idx_hbm.at[j, wid], idxv.at[j])

        def start(u):
            p = u % 2
            handles = []
            for j in range(6):
                tbl = rel_hbm if j in (1, 4) else ent_hbm
                handles.append(
                    pltpu.async_copy(tbl.at[idxv.at[j, u]], rows[p][j], sems[p])
                )
            return handles

        def compute(u, loss_acc):
            p = u % 2

            def ibody(i, carry):
                z = jnp.zeros((_L,), jnp.float32)
                accp, accn = z, z
                for kk in range(_D // _L):
                    sl = pl.ds(kk * _L, _L)
                    sp = (
                        _signish(rows[p][0][i, sl])
                        + rows[p][1][i, sl]
                        - _signish(rows[p][2][i, sl])
                    )
                    sn = (
                        _signish(rows[p][3][i, sl])
                        + rows[p][4][i, sl]
                        - _signish(rows[p][5][i, sl])
                    )
                    accp = accp + sp * sp
                    accn = accn + sn * sn
                term = jnp.maximum(
                    _vsqrt(jnp.sum(accp)) - _vsqrt(jnp.sum(accn)) + 1.0, 0.0
                )
                return carry + term

            return lax.fori_loop(0, _C, ibody, loss_acc, unroll=4)

        copies = start(0)
        loss = jnp.float32(0.0)
        for u in range(_NCH):
            for h in copies:
                h.wait()
            copies = start(u + 1) if u + 1 < _NCH else []
            loss = compute(u, loss)

        lane = lax.iota(jnp.int32, _L)
        lossbuf[...] = jnp.where(lane == 0, loss, jnp.float32(0.0))
        pltpu.sync_copy(lossbuf, out_hbm.at[pl.ds(wid * _L, _L)])

    return k(entity_emb, relation_emb, idx)


def _tc_pad(t):
    n, d = t.shape
    r = 2000

    def body(x_ref, o_ref):
        o_ref[...] = jnp.concatenate(
            [x_ref[...], jnp.zeros((r, _DP - _D), jnp.float32)], axis=1
        )

    return pl.pallas_call(
        body,
        grid=(n // r,),
        in_specs=[pl.BlockSpec((r, d), lambda i: (i, 0))],
        out_specs=pl.BlockSpec((r, _DP), lambda i: (i, 0)),
        out_shape=jax.ShapeDtypeStruct((n, _DP), jnp.float32),
    )(t)


def _tc_reduce(partials):
    def body(x_ref, o_ref):
        o_ref[...] = jnp.full((1, 1), jnp.sum(x_ref[...]))

    return pl.pallas_call(
        body,
        out_shape=jax.ShapeDtypeStruct((1, 1), jnp.float32),
    )(partials)


def kernel(pos_exmpls, neg_exmpls, entity_emb, relation_emb):
    pos = pos_exmpls.astype(jnp.int32)
    neg = neg_exmpls.astype(jnp.int32)
    idx = jnp.concatenate([pos.T, neg.T], axis=0).reshape(6, _NW, _NCH, _C)
    ent_p = _tc_pad(entity_emb)
    rel_p = _tc_pad(relation_emb)
    partials = _sc_partials(ent_p, rel_p, idx)
    return _tc_reduce(partials)[0, 0]


# speedup vs baseline: 1.6084x; 1.6084x over previous
import functools

import jax
import jax.numpy as jnp
from jax import lax
from jax.experimental import pallas as pl
from jax.experimental.pallas import tpu as pltpu
from jax.experimental.pallas import tpu_sc as plsc

_B = 16384
_D = 64
_DP = 128
_L = 16
_NC = 2
_NS = 16
_NW = _NC * _NS
_TB = _B // _NW
_C = 64
_NCH = _TB // _C


def _vsqrt(x):
    xi = lax.bitcast_convert_type(x, jnp.int32)
    yi = jnp.int32(0x5F3759DF) - (xi >> 1)
    y = lax.bitcast_convert_type(yi, jnp.float32)
    for _ in range(3):
        y = y * (1.5 - 0.5 * x * y * y)
    return x * y


def _signish(x):
    return jnp.minimum(jnp.maximum(x * 1e12, -1.0), 1.0)


def _sc_partials(entity_emb, relation_emb, idx):
    mesh = plsc.VectorSubcoreMesh(core_axis_name="c", subcore_axis_name="s")

    @functools.partial(
        pl.kernel,
        mesh=mesh,
        compiler_params=pltpu.CompilerParams(
            needs_layout_passes=False, use_tc_tiling_on_sc=True
        ),
        out_type=jax.ShapeDtypeStruct((_NW * _L,), jnp.float32),
        scratch_types=[pltpu.VMEM((6, _NCH, _C), jnp.int32)]
        + [pltpu.VMEM((_C, _DP), jnp.float32) for _ in range(12)]
        + [
            pltpu.VMEM((_L,), jnp.float32),
            pltpu.SemaphoreType.DMA,
            pltpu.SemaphoreType.DMA,
        ],
    )
    def k(ent_hbm, rel_hbm, idx_hbm, out_hbm, idxv, *rest):
        rows = [rest[0:6], rest[6:12]]
        lossbuf, sem0, sem1 = rest[12], rest[13], rest[14]
        wid = lax.axis_index("s") * _NC + lax.axis_index("c")
        sems = [sem0, sem1]

        for j in range(6):
            pltpu.sync_copy(idx_hbm.at[j, wid], idxv.at[j])

        def start(u):
            p = u % 2
            handles = []
            for j in range(6):
                tbl = rel_hbm if j in (1, 4) else ent_hbm
                handles.append(
                    pltpu.async_copy(tbl.at[idxv.at[j, u]], rows[p][j], sems[p])
                )
            return handles

        def compute(u, loss_acc):
            p = u % 2

            def ibody(i, carry):
                z = jnp.zeros((_L,), jnp.float32)
                accp, accn = z, z
                for kk in range(_D // _L):
                    sl = pl.ds(kk * _L, _L)
                    sp = (
                        _signish(rows[p][0][i, sl])
                        + rows[p][1][i, sl]
                        - _signish(rows[p][2][i, sl])
                    )
                    sn = (
                        _signish(rows[p][3][i, sl])
                        + rows[p][4][i, sl]
                        - _signish(rows[p][5][i, sl])
                    )
                    accp = accp + sp * sp
                    accn = accn + sn * sn
                term = jnp.maximum(
                    _vsqrt(jnp.sum(accp)) - _vsqrt(jnp.sum(accn)) + 1.0, 0.0
                )
                return carry + term

            return lax.fori_loop(0, _C, ibody, loss_acc, unroll=4)

        copies = start(0)
        loss = jnp.float32(0.0)
        for u in range(_NCH):
            for h in copies:
                h.wait()
            copies = start(u + 1) if u + 1 < _NCH else []
            loss = compute(u, loss)

        lane = lax.iota(jnp.int32, _L)
        lossbuf[...] = jnp.where(lane == 0, loss, jnp.float32(0.0))
        pltpu.sync_copy(lossbuf, out_hbm.at[pl.ds(wid * _L, _L)])

    return k(entity_emb, relation_emb, idx)


def _tc_trans_pad(t_t):
    d, n = t_t.shape
    r = 2000

    def body(x_ref, o_ref):
        x = x_ref[...]
        rr = lax.broadcasted_iota(jnp.int32, (d, d), 0)
        cc = lax.broadcasted_iota(jnp.int32, (d, d), 1)
        eye = (rr == cc).astype(jnp.float32)
        xt = lax.dot_general(
            x,
            eye,
            (((0,), (0,)), ((), ())),
            preferred_element_type=jnp.float32,
            precision=lax.Precision.HIGHEST,
        )
        o_ref[...] = jnp.concatenate(
            [xt, jnp.zeros((r, _DP - _D), jnp.float32)], axis=1
        )

    return pl.pallas_call(
        body,
        grid=(n // r,),
        in_specs=[pl.BlockSpec((d, r), lambda i: (0, i))],
        out_specs=pl.BlockSpec((r, _DP), lambda i: (i, 0)),
        out_shape=jax.ShapeDtypeStruct((n, _DP), jnp.float32),
    )(t_t)


def _tc_reduce(partials):
    def body(x_ref, o_ref):
        o_ref[...] = jnp.full((1, 1), jnp.sum(x_ref[...]))

    return pl.pallas_call(
        body,
        out_shape=jax.ShapeDtypeStruct((1, 1), jnp.float32),
    )(partials)


def kernel(pos_exmpls, neg_exmpls, entity_emb, relation_emb):
    pos = pos_exmpls.astype(jnp.int32)
    neg = neg_exmpls.astype(jnp.int32)
    idx = jnp.concatenate([pos.T, neg.T], axis=0).reshape(6, _NW, _NCH, _C)
    ent_p = jnp.pad(entity_emb, ((0, 0), (0, _DP - _D)))
    rel_p = jnp.pad(relation_emb, ((0, 0), (0, _DP - _D)))
    partials = _sc_partials(ent_p, rel_p, idx)
    return _tc_reduce(partials)[0, 0]
